# baseline (speedup 1.0000x reference)
import functools
import jax
import jax.numpy as jnp
from jax import lax
from jax.experimental import pallas as pl
from jax.experimental.pallas import tpu as pltpu, tpu_sc as plsc

C_ROWS = 16
UNROLL = 8


def kernel(x, emb_weight):
    B, T, D = x.shape
    NW = 32
    rows_w = T // NW
    n_chunks = rows_w // C_ROWS
    n_grp = (C_ROWS * D) // (16 * UNROLL)
    grp_per_row = D // (16 * UNROLL)
    n_steps = n_chunks * B

    x2 = x.reshape(B * T, D)
    emb2 = emb_weight[:T]
    mesh = plsc.VectorSubcoreMesh(core_axis_name="c", subcore_axis_name="s")

    @functools.partial(
        pl.kernel,
        mesh=mesh,
        out_type=jax.ShapeDtypeStruct((B * T, D), jnp.float32),
        scratch_types=[
            pltpu.VMEM((2, C_ROWS, D), jnp.float32),
            pltpu.VMEM((2, C_ROWS, D), jnp.float32),
            pltpu.SemaphoreType.DMA((2,)),
            pltpu.SemaphoreType.DMA((2,)),
            pltpu.SemaphoreType.DMA((2,)),
        ],
        compiler_params=pltpu.CompilerParams(use_tc_tiling_on_sc=True),
    )
    def k(x_hbm, emb_hbm, out_hbm, xb, eb, sx, se, st):
        wid = lax.axis_index("s") * 2 + lax.axis_index("c")
        t0 = wid * rows_w

        handles = {}

        def row0(s):
            ci, b = divmod(s, B)
            return b * T + t0 + ci * C_ROWS, ci

        def load_step(s):
            r0, ci = row0(s)
            handles[("x", s)] = pltpu.async_copy(
                x_hbm.at[pl.ds(r0, C_ROWS)], xb.at[s % 2], sx.at[s % 2])
            if s % B == 0:
                p = ci % 2
                handles[("e", ci)] = pltpu.async_copy(
                    emb_hbm.at[pl.ds(t0 + ci * C_ROWS, C_ROWS)], eb.at[p],
                    se.at[p])

        load_step(0)
        for s in range(n_steps):
            q = s % 2
            if s + 1 < n_steps:
                if s >= 1:
                    handles[("st", s - 1)].wait()
                load_step(s + 1)
            r0, ci = row0(s)
            p = ci % 2
            if s % B == 0:
                handles[("e", ci)].wait()
            handles[("x", s)].wait()

            def add_body(g, _):
                r = g // grp_per_row
                c0 = (g % grp_per_row) * (16 * UNROLL)
                vals = [eb[p, r, pl.ds(c0 + u * 16, 16)]
                        for u in range(UNROLL)]
                for u in range(UNROLL):
                    plsc.addupdate(
                        xb.at[q, r, pl.ds(c0 + u * 16, 16)], vals[u])
                return 0

            lax.fori_loop(0, n_grp, add_body, 0)
            handles[("st", s)] = pltpu.async_copy(
                xb.at[q], out_hbm.at[pl.ds(r0, C_ROWS)], st.at[q])
        handles[("st", n_steps - 2)].wait()
        handles[("st", n_steps - 1)].wait()

    out = k(x2, emb2)
    return out.reshape(B, T, D)


# speedup vs baseline: 1.1223x; 1.1223x over previous
import functools
import jax
import jax.numpy as jnp
from jax import lax
from jax.experimental import pallas as pl
from jax.experimental.pallas import tpu as pltpu, tpu_sc as plsc

C_ROWS = 8
UNROLL = 8


def kernel(x, emb_weight):
    B, T, D = x.shape
    NW = 32
    rows_w = T // NW
    n_chunks = rows_w // C_ROWS
    n_grp = (C_ROWS * D) // (16 * UNROLL)
    grp_per_row = D // (16 * UNROLL)

    x2 = x.reshape(B * T, D)
    emb2 = emb_weight[:T]
    mesh = plsc.VectorSubcoreMesh(core_axis_name="c", subcore_axis_name="s")

    @functools.partial(
        pl.kernel,
        mesh=mesh,
        out_type=jax.ShapeDtypeStruct((B * T, D), jnp.float32),
        scratch_types=[
            pltpu.VMEM((3, B, C_ROWS, D), jnp.float32),
            pltpu.VMEM((3, C_ROWS, D), jnp.float32),
            pltpu.SemaphoreType.DMA((3, B)),
            pltpu.SemaphoreType.DMA((3,)),
            pltpu.SemaphoreType.DMA((3, B)),
        ],
        compiler_params=pltpu.CompilerParams(use_tc_tiling_on_sc=True),
    )
    def k(x_hbm, emb_hbm, out_hbm, xb, eb, sx, se, st):
        wid = lax.axis_index("s") * 2 + lax.axis_index("c")
        t0 = wid * rows_w

        handles = {}

        def load_chunk(ci):
            p = ci % 3
            r0 = t0 + ci * C_ROWS
            handles[("e", ci)] = pltpu.async_copy(
                emb_hbm.at[pl.ds(r0, C_ROWS)], eb.at[p], se.at[p])
            for b in range(B):
                handles[("x", ci, b)] = pltpu.async_copy(
                    x_hbm.at[pl.ds(b * T + r0, C_ROWS)], xb.at[p, b],
                    sx.at[p, b])

        load_chunk(0)
        load_chunk(1)
        for ci in range(n_chunks):
            p = ci % 3
            if ci + 2 < n_chunks:
                if ci >= 1:
                    for b in range(B):
                        handles[("s", ci - 1, b)].wait()
                load_chunk(ci + 2)
            handles[("e", ci)].wait()
            for b in range(B):
                handles[("x", ci, b)].wait()

                def add_body(g, _):
                    r = g // grp_per_row
                    c0 = (g % grp_per_row) * (16 * UNROLL)
                    vals = [eb[p, r, pl.ds(c0 + u * 16, 16)]
                            for u in range(UNROLL)]
                    for u in range(UNROLL):
                        plsc.addupdate(
                            xb.at[p, b, r, pl.ds(c0 + u * 16, 16)], vals[u])
                    return 0

                lax.fori_loop(0, n_grp, add_body, 0)
                r0 = t0 + ci * C_ROWS
                handles[("s", ci, b)] = pltpu.async_copy(
                    xb.at[p, b], out_hbm.at[pl.ds(b * T + r0, C_ROWS)],
                    st.at[p, b])
        for b in range(B):
            handles[("s", n_chunks - 3, b)].wait()
            handles[("s", n_chunks - 2, b)].wait()
            handles[("s", n_chunks - 1, b)].wait()

    out = k(x2, emb2)
    return out.reshape(B, T, D)
